# baseline (speedup 1.0000x reference)
import functools
import math

import jax
import jax.numpy as jnp
from jax import lax
from jax.experimental import pallas as pl
from jax.experimental.pallas import tpu as pltpu
from jax.experimental.pallas import tpu_sc as plsc

D_MODEL = 64
SCALE = math.sqrt(D_MODEL)
LANES = 16

NUM_CORES = 2
NUM_SUBCORES = 16
NW = NUM_CORES * NUM_SUBCORES

SEQ = 200
BATCH = 4096
BW = BATCH // NW
NBUF = 4
FIRE_AHEAD = 2

_mesh = plsc.VectorSubcoreMesh(core_axis_name="c", subcore_axis_name="s")


@functools.partial(
    pl.kernel,
    out_type=jax.ShapeDtypeStruct((SEQ, D_MODEL, BATCH), jnp.float32),
    mesh=_mesh,
    scratch_types=[
        pltpu.VMEM((SEQ, BW), jnp.int32),
        [pltpu.VMEM((BW, D_MODEL), jnp.float32) for _ in range(NBUF)],
        [pltpu.VMEM((D_MODEL, BW), jnp.float32) for _ in range(NBUF)],
        [pltpu.SemaphoreType.DMA for _ in range(NBUF)],
        [pltpu.SemaphoreType.DMA for _ in range(NBUF)],
    ],
    compiler_params=pltpu.CompilerParams(
        use_tc_tiling_on_sc=False, needs_layout_passes=False
    ),
)
def _emb_lookup(xt_hbm, table_hbm, out_hbm, idx_v, rows, trans, sem_g, sem_s):
    wid = lax.axis_index("s") * NUM_CORES + lax.axis_index("c")
    bbase = wid * BW

    pltpu.sync_copy(xt_hbm.at[:, pl.ds(bbase, BW)], idx_v)

    lane = lax.iota(jnp.int32, LANES)

    def fire_gather(t, b):
        pltpu.async_copy(table_hbm.at[idx_v.at[t]], rows[b], sem_g[b])

    def wait_gather(b):
        pltpu.make_async_copy(table_hbm.at[idx_v.at[0]], rows[b], sem_g[b]).wait()

    def wait_store(b):
        pltpu.make_async_copy(
            trans[b], out_hbm.at[0, :, pl.ds(0, BW)], sem_s[b]
        ).wait()

    for t in range(FIRE_AHEAD):
        fire_gather(t, t)

    def outer(t0, carry):
        for b in range(NBUF):
            t = t0 * NBUF + b
            fb = (b + FIRE_AHEAD) % NBUF

            @pl.when(t + FIRE_AHEAD < SEQ)
            def _():
                fire_gather(t + FIRE_AHEAD, fb)

            wait_gather(b)

            @pl.when(t >= NBUF)
            def _():
                wait_store(b)

            @plsc.parallel_loop(0, D_MODEL, unroll=4)
            def _(c):
                col = jnp.full((LANES,), c, dtype=jnp.int32)
                for j in range(BW // LANES):
                    v = plsc.load_gather(rows[b], [lane + (j * LANES), col])
                    trans[b][c, pl.ds(j * LANES, LANES)] = v * SCALE

            pltpu.async_copy(
                trans[b], out_hbm.at[t, :, pl.ds(bbase, BW)], sem_s[b]
            )
        return carry

    lax.fori_loop(0, SEQ // NBUF, outer, 0)

    for b in range(NBUF):
        wait_store(b)


def kernel(x, table):
    xt = jnp.transpose(x.astype(jnp.int32))
    out = _emb_lookup(xt, table)
    return jnp.transpose(out, (2, 0, 1))


# speedup vs baseline: 1.5265x; 1.5265x over previous
import functools
import math

import jax
import jax.numpy as jnp
from jax import lax
from jax.experimental import pallas as pl
from jax.experimental.pallas import tpu as pltpu
from jax.experimental.pallas import tpu_sc as plsc

D_MODEL = 64
SCALE = math.sqrt(D_MODEL)
LANES = 16

NUM_CORES = 2
NUM_SUBCORES = 16
NW = NUM_CORES * NUM_SUBCORES

SEQ = 200
BATCH = 4096
BW = BATCH // NW
BW_PAD = BW + 1
NBUF = 4
FIRE_AHEAD = 2

_mesh = plsc.VectorSubcoreMesh(core_axis_name="c", subcore_axis_name="s")


@functools.partial(
    pl.kernel,
    out_type=jax.ShapeDtypeStruct((SEQ, D_MODEL, BATCH), jnp.float32),
    mesh=_mesh,
    scratch_types=[
        pltpu.VMEM((SEQ, BW), jnp.int32),
        [pltpu.VMEM((BW, D_MODEL), jnp.float32) for _ in range(NBUF)],
        [pltpu.VMEM((D_MODEL, BW_PAD), jnp.float32) for _ in range(NBUF)],
        [pltpu.SemaphoreType.DMA for _ in range(NBUF)],
        [pltpu.SemaphoreType.DMA for _ in range(NBUF)],
    ],
    compiler_params=pltpu.CompilerParams(
        use_tc_tiling_on_sc=False, needs_layout_passes=False
    ),
)
def _emb_lookup(xt_hbm, table_hbm, out_hbm, idx_v, rows, trans, sem_g, sem_s):
    wid = lax.axis_index("s") * NUM_CORES + lax.axis_index("c")
    bbase = wid * BW

    pltpu.sync_copy(xt_hbm.at[:, pl.ds(bbase, BW)], idx_v)

    lane = lax.iota(jnp.int32, LANES)

    def fire_gather(t, b):
        pltpu.async_copy(table_hbm.at[idx_v.at[t]], rows[b], sem_g[b])

    def wait_gather(b):
        pltpu.make_async_copy(table_hbm.at[idx_v.at[0]], rows[b], sem_g[b]).wait()

    def wait_store(b):
        pltpu.make_async_copy(
            trans[b].at[:, pl.ds(0, BW)], out_hbm.at[0, :, pl.ds(0, BW)], sem_s[b]
        ).wait()

    for t in range(FIRE_AHEAD):
        fire_gather(t, t)

    def outer(t0, carry):
        for b in range(NBUF):
            t = t0 * NBUF + b
            fb = (b + FIRE_AHEAD) % NBUF

            @pl.when(t + FIRE_AHEAD < SEQ)
            def _():
                fire_gather(t + FIRE_AHEAD, fb)

            wait_gather(b)

            @pl.when(t >= NBUF)
            def _():
                wait_store(b)

            @plsc.parallel_loop(0, BW, unroll=4)
            def _(i):
                coli = jnp.full((LANES,), i, dtype=jnp.int32)
                for q in range(D_MODEL // LANES):
                    v = rows[b][i, pl.ds(q * LANES, LANES)] * SCALE
                    plsc.store_scatter(trans[b], [lane + (q * LANES), coli], v)

            pltpu.async_copy(
                trans[b].at[:, pl.ds(0, BW)],
                out_hbm.at[t, :, pl.ds(bbase, BW)],
                sem_s[b],
            )
        return carry

    lax.fori_loop(0, SEQ // NBUF, outer, 0)

    for b in range(NBUF):
        wait_store(b)


def kernel(x, table):
    xt = jnp.transpose(x.astype(jnp.int32))
    out = _emb_lookup(xt, table)
    return jnp.transpose(out, (2, 0, 1))
